# Initial kernel scaffold; baseline (speedup 1.0000x reference)
import jax
import jax.numpy as jnp
from jax.experimental import pallas as pl


def kernel(x, edge_index, edge_attr, A, B, C, D, Ew):
    raise NotImplementedError("write your pallas kernel here")


# speedup vs baseline: 1.8890x; 1.8890x over previous
import functools

import jax
import jax.numpy as jnp
from jax import lax
from jax.experimental import pallas as pl
from jax.experimental.pallas import tpu as pltpu
from jax.experimental.pallas import tpu_sc as plsc

NC = 2
NS = 16
L = 16

HALF = 64


def _tables_body(x_ref, b_ref, d_ref, ew_ref, tsrc_ref, tdst_ref):
    xb = x_ref[...]
    xB = jnp.dot(xb, b_ref[...], preferred_element_type=jnp.float32)
    xD = jnp.dot(xb, d_ref[...], preferred_element_type=jnp.float32)
    xE = jnp.dot(xb, ew_ref[...], preferred_element_type=jnp.float32)
    tsrc_ref[0] = jnp.concatenate([xE[:, :HALF], xB[:, :HALF]], axis=1)
    tsrc_ref[1] = jnp.concatenate([xE[:, HALF:], xB[:, HALF:]], axis=1)
    z = jnp.zeros_like(xD[:, :HALF])
    tdst_ref[0] = jnp.concatenate([xD[:, :HALF], z], axis=1)
    tdst_ref[1] = jnp.concatenate([xD[:, HALF:], z], axis=1)


def _make_tables(x, B, D, Ew):
    N, d = x.shape
    NB = 2000
    grid = (N // NB,)
    return pl.pallas_call(
        _tables_body,
        grid=grid,
        in_specs=[
            pl.BlockSpec((NB, d), lambda i: (i, 0)),
            pl.BlockSpec((d, d), lambda i: (0, 0)),
            pl.BlockSpec((d, d), lambda i: (0, 0)),
            pl.BlockSpec((d, d), lambda i: (0, 0)),
        ],
        out_specs=[
            pl.BlockSpec((2, NB, d), lambda i: (0, i, 0)),
            pl.BlockSpec((2, NB, d), lambda i: (0, i, 0)),
        ],
        out_shape=[
            jax.ShapeDtypeStruct((2, N, d), jnp.float32),
            jax.ShapeDtypeStruct((2, N, d), jnp.float32),
        ],
    )(x, B, D, Ew)


def _ec_body(ea_ref, c_ref, out_ref):
    ev = jnp.dot(ea_ref[...], c_ref[...], preferred_element_type=jnp.float32)
    out_ref[0] = ev[:, :HALF]
    out_ref[1] = ev[:, HALF:]


def _make_ec(edge_attr, C):
    E, de = edge_attr.shape
    d = C.shape[1]
    EB = 8000
    return pl.pallas_call(
        _ec_body,
        grid=(E // EB,),
        in_specs=[
            pl.BlockSpec((EB, de), lambda i: (i, 0)),
            pl.BlockSpec((de, d), lambda i: (0, 0)),
        ],
        out_specs=pl.BlockSpec((2, EB, HALF), lambda i: (0, i, 0)),
        out_shape=jax.ShapeDtypeStruct((2, E, HALF), jnp.float32),
    )(edge_attr, C)


def _sc_gate(N, E, BLK):
    EPT = E // NS
    NBLK = EPT // BLK
    d = 2 * HALF

    N1 = 7504
    N2 = N - N1
    AR = 7600
    ZCH = 200
    FCH1 = 56
    FCH2 = 48

    mesh = plsc.VectorSubcoreMesh(core_axis_name="c", subcore_axis_name="s")

    @functools.partial(
        pl.kernel,
        mesh=mesh,
        out_type=[
            jax.ShapeDtypeStruct((NC * N, d), jnp.float32),
            jax.ShapeDtypeStruct((NC * E, d), jnp.float32),
        ],
        scratch_types=[
            pltpu.VMEM((BLK,), jnp.int32),
            pltpu.VMEM((BLK,), jnp.int32),
            pltpu.VMEM((BLK,), jnp.int32),
            pltpu.VMEM((BLK,), jnp.int32),
            pltpu.VMEM((BLK,), jnp.int32),
            pltpu.VMEM((BLK, HALF), jnp.float32),
            pltpu.VMEM((BLK, d), jnp.float32),
            pltpu.VMEM((BLK, d), jnp.float32),
            pltpu.VMEM((BLK, d), jnp.float32),
            pltpu.VMEM((ZCH, d), jnp.float32),
            pltpu.VMEM_SHARED((AR, d), jnp.float32),
            pltpu.SemaphoreType.DMA,
            pltpu.SemaphoreType.DMA,
        ],
    )
    def k(src_hbm, dst_hbm, tsrc_hbm, tdst_hbm, ec_hbm, out_hbm, spill_hbm,
          sidx, didx, gsidx, gdidx, qidx, ecv, tsv, tdv, msd, zbuf, acc,
          sem1, sem2):
        c = lax.axis_index("c")
        s = lax.axis_index("s")
        coff = c * N

        zv = jnp.zeros((L,), jnp.float32)

        def zloop(t, _):
            i = t // (d // L)
            j = t % (d // L)
            zbuf[i, pl.ds(j * L, L)] = zv
            return 0

        lax.fori_loop(0, ZCH * (d // L), zloop, 0)
        for kk in range(-(-(AR // ZCH) // NS)):
            ch = s + kk * NS
            @pl.when(ch < AR // ZCH)
            def _():
                pltpu.sync_copy(zbuf, acc.at[pl.ds(ch * ZCH, ZCH)])
        plsc.subcore_barrier()

        def block1(blk, _):
            base = s * EPT + blk * BLK
            pltpu.sync_copy(src_hbm.at[pl.ds(base, BLK)], sidx)
            pltpu.sync_copy(dst_hbm.at[pl.ds(base, BLK)], didx)
            for kk in range(BLK // L):
                sl = pl.ds(kk * L, L)
                gsidx[sl] = sidx[sl] + coff
                gdidx[sl] = didx[sl] + coff
                dv = didx[sl]
                qidx[sl] = jnp.where(dv < N1, dv, N1)
            g1 = pltpu.async_copy(tsrc_hbm.at[gsidx], tsv, sem1)
            g2 = pltpu.async_copy(tdst_hbm.at[gdidx], tdv, sem2)
            pltpu.sync_copy(ec_hbm.at[pl.ds(c * E + base, BLK)], ecv)
            g1.wait()
            g2.wait()

            def edge(j, _):
                for q in range(HALF // L):
                    sl = pl.ds(q * L, L)
                    sh = pl.ds(HALF + q * L, L)
                    e = ecv[j, sl] + tdv[j, sl] + tsv[j, sl]
                    sg = 1.0 / (1.0 + jnp.exp(-e))
                    msd[j, sl] = sg * tsv[j, sh]
                    msd[j, sh] = sg
                return 0

            lax.fori_loop(0, BLK, edge, 0)
            pltpu.sync_copy(msd, acc.at[qidx], add=True)
            pltpu.sync_copy(msd, spill_hbm.at[pl.ds(c * E + base, BLK)])
            return 0

        lax.fori_loop(0, NBLK, block1, 0)

        plsc.subcore_barrier()
        for kk in range(-(-(N1 // FCH1) // NS)):
            ch = s + kk * NS
            @pl.when(ch < N1 // FCH1)
            def _():
                pltpu.sync_copy(acc.at[pl.ds(ch * FCH1, FCH1)],
                                out_hbm.at[pl.ds(coff + ch * FCH1, FCH1)])
        plsc.subcore_barrier()

        @pl.when(s < 13)
        def _():
            pltpu.sync_copy(zbuf, acc.at[pl.ds(s * ZCH, ZCH)])
        plsc.subcore_barrier()

        def block2(blk, _):
            base = s * EPT + blk * BLK
            pltpu.sync_copy(dst_hbm.at[pl.ds(base, BLK)], didx)
            for kk in range(BLK // L):
                sl = pl.ds(kk * L, L)
                dv = didx[sl] - N1
                qidx[sl] = jnp.where(dv >= 0, dv, N2)
            pltpu.sync_copy(spill_hbm.at[pl.ds(c * E + base, BLK)], msd)
            pltpu.sync_copy(msd, acc.at[qidx], add=True)
            return 0

        lax.fori_loop(0, NBLK, block2, 0)

        plsc.subcore_barrier()
        for kk in range(-(-(N2 // FCH2) // NS)):
            ch = s + kk * NS
            @pl.when(ch < N2 // FCH2)
            def _():
                pltpu.sync_copy(acc.at[pl.ds(ch * FCH2, FCH2)],
                                out_hbm.at[pl.ds(coff + N1 + ch * FCH2, FCH2)])

    return k


def _epi_body(x_ref, a_ref, a0_ref, a1_ref, out_ref):
    xb = x_ref[...]
    a0 = a0_ref[...]
    a1 = a1_ref[...]
    num = jnp.concatenate([a0[:, :HALF], a1[:, :HALF]], axis=1)
    den = jnp.concatenate([a0[:, HALF:], a1[:, HALF:]], axis=1) + 1e-6
    h = jnp.dot(xb, a_ref[...], preferred_element_type=jnp.float32)
    h = h + num / den
    out_ref[...] = xb + jnp.maximum(h, 0.0)


def _epilogue(x, A, accd):
    N, d = x.shape
    NB = 2000
    nb = N // NB
    return pl.pallas_call(
        _epi_body,
        grid=(nb,),
        in_specs=[
            pl.BlockSpec((NB, d), lambda i: (i, 0)),
            pl.BlockSpec((d, d), lambda i: (0, 0)),
            pl.BlockSpec((NB, d), lambda i: (i, 0)),
            pl.BlockSpec((NB, d), lambda i, nb=nb: (i + nb, 0)),
        ],
        out_specs=pl.BlockSpec((NB, d), lambda i: (i, 0)),
        out_shape=jax.ShapeDtypeStruct((N, d), jnp.float32),
    )(x, A, accd, accd)


def kernel(x, edge_index, edge_attr, A, B, C, D, Ew):
    N, d = x.shape
    E = edge_index.shape[1]
    src = edge_index[0]
    dst = edge_index[1]

    tsrc, tdst = _make_tables(x, B, D, Ew)
    ec = _make_ec(edge_attr, C)
    tsrc = tsrc.reshape(NC * N, d)
    tdst = tdst.reshape(NC * N, d)
    ec = ec.reshape(NC * E, HALF)

    accd, _spill = _sc_gate(N, E, BLK=80)(src, dst, tsrc, tdst, ec)
    return _epilogue(x, A, accd)
